# baseline (speedup 1.0000x reference)
import functools

import jax
import jax.numpy as jnp
from jax import lax
from jax.experimental import pallas as pl
from jax.experimental.pallas import tpu as pltpu
from jax.experimental.pallas import tpu_sc as plsc

NUM_KV_HEADS = 8
KV_HEAD_DIM = 64
KV_DIM = NUM_KV_HEADS * KV_HEAD_DIM

_info = plsc.get_sparse_core_info()
NC = _info.num_cores
NS = _info.num_subcores
NW = NC * NS

CHUNK = 128


def _ve_body(ids_hbm, w0_hbm, w1_hbm, w2_hbm, out_hbm, idx_v, rows_v, sem):
    n_ids = ids_hbm.shape[0]
    rows_per_w = n_ids // NW
    wid = lax.axis_index("s") * NC + lax.axis_index("c")
    base = wid * rows_per_w
    pltpu.sync_copy(ids_hbm.at[pl.ds(base, rows_per_w)], idx_v)
    for l, w_hbm in enumerate((w0_hbm, w1_hbm, w2_hbm)):
        for c in range(rows_per_w // CHUNK):
            idx_slice = idx_v.at[pl.ds(c * CHUNK, CHUNK)]
            pltpu.async_copy(w_hbm.at[idx_slice], rows_v, sem).wait()
            pltpu.sync_copy(
                rows_v, out_hbm.at[l, pl.ds(base + c * CHUNK, CHUNK)]
            )


@jax.jit
def kernel(input_ids, w0, w1, w2):
    b, t = input_ids.shape
    n_ids = b * t
    rows_per_w = n_ids // NW
    ids_flat = input_ids.reshape(n_ids)
    mesh = plsc.VectorSubcoreMesh(core_axis_name="c", subcore_axis_name="s")
    out = pl.kernel(
        _ve_body,
        out_type=jax.ShapeDtypeStruct((3, n_ids, KV_DIM), jnp.float32),
        mesh=mesh,
        scratch_types=[
            pltpu.VMEM((rows_per_w,), jnp.int32),
            pltpu.VMEM((CHUNK, KV_DIM), jnp.float32),
            pltpu.SemaphoreType.DMA,
        ],
    )(ids_flat, w0, w1, w2)
    return out.reshape(3, b, t, NUM_KV_HEADS, KV_HEAD_DIM)


# speedup vs baseline: 1.0076x; 1.0076x over previous
import functools

import jax
import jax.numpy as jnp
from jax import lax
from jax.experimental import pallas as pl
from jax.experimental.pallas import tpu as pltpu
from jax.experimental.pallas import tpu_sc as plsc

NUM_KV_HEADS = 8
KV_HEAD_DIM = 64
KV_DIM = NUM_KV_HEADS * KV_HEAD_DIM

_info = plsc.get_sparse_core_info()
NC = _info.num_cores
NS = _info.num_subcores
NW = NC * NS

CHUNK = 64
NBUF = 2


def _ve_body(
    ids_hbm, w0_hbm, w1_hbm, w2_hbm, out_hbm,
    idx_v, rows0, rows1, sem_in0, sem_in1, sem_out0, sem_out1,
):
    n_ids = ids_hbm.shape[0]
    rows_per_w = n_ids // NW
    n_chunks = rows_per_w // CHUNK
    wid = lax.axis_index("s") * NC + lax.axis_index("c")
    base = wid * rows_per_w
    pltpu.sync_copy(ids_hbm.at[pl.ds(base, rows_per_w)], idx_v)

    rows = (rows0, rows1)
    sem_in = (sem_in0, sem_in1)
    sem_out = (sem_out0, sem_out1)

    steps = [
        (w_hbm, l, c)
        for l, w_hbm in enumerate((w0_hbm, w1_hbm, w2_hbm))
        for c in range(n_chunks)
    ]
    n = len(steps)

    def start_gather(i):
        w_hbm, _, c = steps[i]
        b = i % NBUF
        idx_slice = idx_v.at[pl.ds(c * CHUNK, CHUNK)]
        return pltpu.async_copy(w_hbm.at[idx_slice], rows[b], sem_in[b])

    def start_write(i):
        _, l, c = steps[i]
        b = i % NBUF
        return pltpu.async_copy(
            rows[b], out_hbm.at[l, pl.ds(base + c * CHUNK, CHUNK)], sem_out[b]
        )

    gathers = {0: start_gather(0)}
    writes = {}
    for i in range(n):
        gathers.pop(i).wait()
        writes[i] = start_write(i)
        if i + 1 < n:
            if i >= 1:
                writes.pop(i - 1).wait()
            gathers[i + 1] = start_gather(i + 1)
    for i in sorted(writes):
        writes[i].wait()


@jax.jit
def kernel(input_ids, w0, w1, w2):
    b, t = input_ids.shape
    n_ids = b * t
    rows_per_w = n_ids // NW
    ids_flat = input_ids.reshape(n_ids)
    mesh = plsc.VectorSubcoreMesh(core_axis_name="c", subcore_axis_name="s")
    out = pl.kernel(
        _ve_body,
        out_type=jax.ShapeDtypeStruct((3, n_ids, KV_DIM), jnp.float32),
        mesh=mesh,
        scratch_types=[
            pltpu.VMEM((rows_per_w,), jnp.int32),
            pltpu.VMEM((CHUNK, KV_DIM), jnp.float32),
            pltpu.VMEM((CHUNK, KV_DIM), jnp.float32),
            pltpu.SemaphoreType.DMA,
            pltpu.SemaphoreType.DMA,
            pltpu.SemaphoreType.DMA,
            pltpu.SemaphoreType.DMA,
        ],
    )(ids_flat, w0, w1, w2)
    return out.reshape(3, b, t, NUM_KV_HEADS, KV_HEAD_DIM)
